# baseline (speedup 1.0000x reference)
import functools

import jax
import jax.numpy as jnp
from jax import lax
from jax.experimental import pallas as pl
from jax.experimental.pallas import tpu as pltpu

E = 64
K = 2
D = 1024
F = 512
R = 16
N = 256


def _gate_body(x_ref, wa_ref, combine_ref):
    x = x_ref[:]
    wa = wa_ref[:]
    gh = lax.dot_general(x, wa, (((1,), (1,)), ((), ())),
                         preferred_element_type=jnp.float32)
    sq = gh * gh
    er = lax.broadcasted_iota(jnp.int32, (E * R, E), 0)
    ec = lax.broadcasted_iota(jnp.int32, (E * R, E), 1)
    sel = jnp.where(er // R == ec, 1.0, 0.0).astype(jnp.float32)
    ss = lax.dot_general(sq, sel, (((1,), (0,)), ((), ())),
                         preferred_element_type=jnp.float32, precision=lax.Precision.HIGHEST)
    col = lax.broadcasted_iota(jnp.int32, (N, E), 1)
    m1 = jnp.max(ss, axis=1, keepdims=True)
    a1 = jnp.min(jnp.where(ss == m1, col, E), axis=1, keepdims=True)
    ssm = jnp.where(col == a1, -1.0, ss)
    m2 = jnp.max(ssm, axis=1, keepdims=True)
    a2 = jnp.min(jnp.where(ssm == m2, col, E), axis=1, keepdims=True)
    s1 = jnp.sqrt(m1)
    s2 = jnp.sqrt(m2)
    e2 = jnp.exp(s2 - s1)
    w1 = 1.0 / (1.0 + e2)
    w2 = e2 / (1.0 + e2)
    combine = jnp.where(col == a1, w1, 0.0) + jnp.where(col == a2, w2, 0.0)
    combine_ref[:] = combine.astype(jnp.float32)


def _ffn_body(x_ref, combine_ref, wa_ref, wb_ref, wup_ref, wdn_ref, out_ref):
    e = pl.program_id(0)

    @pl.when(e == 0)
    def _():
        out_ref[:] = jnp.zeros((N, D), jnp.float32)

    x = x_ref[:]
    wa = wa_ref[0]
    wb = wb_ref[0]
    wup = wup_ref[0]
    wdn = wdn_ref[0]
    gh = lax.dot_general(x, wa, (((1,), (1,)), ((), ())),
                         preferred_element_type=jnp.float32, precision=lax.Precision.HIGHEST)
    h1 = lax.dot_general(gh, wb, (((1,), (1,)), ((), ())),
                         preferred_element_type=jnp.float32, precision=lax.Precision.HIGHEST)
    up = lax.dot_general(x, wup, (((1,), (1,)), ((), ())),
                         preferred_element_type=jnp.float32, precision=lax.Precision.HIGHEST)
    h = h1 * (1.0 / (1.0 + jnp.exp(-h1))) * up
    erow = lax.broadcasted_iota(jnp.int32, (E, 1), 0)
    onehot = jnp.where(erow == e, 1.0, 0.0).astype(jnp.float32)
    cc = lax.dot_general(combine_ref[:], onehot, (((1,), (0,)), ((), ())),
                         preferred_element_type=jnp.float32, precision=lax.Precision.HIGHEST)
    h = h * cc
    yb = lax.dot_general(h, wdn, (((1,), (1,)), ((), ())),
                         preferred_element_type=jnp.float32, precision=lax.Precision.HIGHEST)
    out_ref[:] += yb


@jax.jit
def kernel(hidden_states, W_A, W_B, W_up, W_down):
    orig_shape = hidden_states.shape
    x = hidden_states.reshape(N, D)
    wa2 = W_A.reshape(E * R, D)

    combine = pl.pallas_call(
        _gate_body,
        out_shape=jax.ShapeDtypeStruct((N, E), jnp.float32),
    )(x, wa2)

    out = pl.pallas_call(
        _ffn_body,
        grid=(E,),
        in_specs=[
            pl.BlockSpec((N, D), lambda e: (0, 0)),
            pl.BlockSpec((N, E), lambda e: (0, 0)),
            pl.BlockSpec((1, R, D), lambda e: (e, 0, 0)),
            pl.BlockSpec((1, F, R), lambda e: (e, 0, 0)),
            pl.BlockSpec((1, F, D), lambda e: (e, 0, 0)),
            pl.BlockSpec((1, D, F), lambda e: (e, 0, 0)),
        ],
        out_specs=pl.BlockSpec((N, D), lambda e: (0, 0)),
        out_shape=jax.ShapeDtypeStruct((N, D), jnp.float32),
    )(x, combine, W_A, W_B, W_up, W_down)

    return (out.reshape(orig_shape), None)
